# baseline (speedup 1.0000x reference)
import functools

import jax
import jax.numpy as jnp
from jax import lax
from jax.experimental import pallas as pl
from jax.experimental.pallas import tpu as pltpu
from jax.experimental.pallas import tpu_sc as plsc

B = 16384
EMB = 32
E = 8

_CHUNK = 128


def _make_sc_gather(num_rows, emb):
    info = plsc.get_sparse_core_info()
    nw = info.num_cores * info.num_subcores
    b_per_w = num_rows // nw
    n_chunks = b_per_w // _CHUNK
    mesh = plsc.VectorSubcoreMesh(core_axis_name="c", subcore_axis_name="s")

    @functools.partial(
        pl.kernel,
        mesh=mesh,
        compiler_params=pltpu.CompilerParams(use_tc_tiling_on_sc=False),
        out_type=[
            jax.ShapeDtypeStruct((num_rows, emb), jnp.float32),
            jax.ShapeDtypeStruct((num_rows, emb), jnp.float32),
        ],
        scratch_types=[
            pltpu.VMEM((n_chunks, _CHUNK), jnp.int32),
            pltpu.VMEM((n_chunks, _CHUNK), jnp.int32),
            pltpu.VMEM((b_per_w, emb), jnp.float32),
            pltpu.VMEM((b_per_w, emb), jnp.float32),
            pltpu.SemaphoreType.DMA,
        ],
    )
    def gather_kernel(uidx_hbm, iidx_hbm, wu_hbm, hi_hbm, u_out, v_out,
                      uidx_v, iidx_v, u_rows, v_rows, sem):
        wid = lax.axis_index("s") * info.num_cores + lax.axis_index("c")
        base = wid * b_per_w
        pltpu.sync_copy(uidx_hbm.at[pl.ds(wid * n_chunks, n_chunks)], uidx_v)
        pltpu.sync_copy(iidx_hbm.at[pl.ds(wid * n_chunks, n_chunks)], iidx_v)
        copies = []
        for j in range(n_chunks):
            sl = pl.ds(j * _CHUNK, _CHUNK)
            copies.append(
                pltpu.async_copy(wu_hbm.at[uidx_v.at[j]], u_rows.at[sl], sem))
            copies.append(
                pltpu.async_copy(hi_hbm.at[iidx_v.at[j]], v_rows.at[sl], sem))
        for c in copies:
            c.wait()
        pltpu.sync_copy(u_rows, u_out.at[pl.ds(base, b_per_w)])
        pltpu.sync_copy(v_rows, v_out.at[pl.ds(base, b_per_w)])

    return gather_kernel


def _head_kernel(u_ref, v_ref, g_ref, lwu_ref, lwv_ref, linb_ref, selw_ref,
                 selb_ref, a_ref, b_ref, t_ref, out_ref):
    u = u_ref[...]
    v = v_ref[...]
    logit = (jnp.dot(u, lwu_ref[...], preferred_element_type=jnp.float32)
             + jnp.dot(v, lwv_ref[...], preferred_element_type=jnp.float32)
             + linb_ref[0, 0])
    s = (jnp.dot(u, selw_ref[...], preferred_element_type=jnp.float32)
         + selb_ref[...])
    s = s - jnp.max(s, axis=1, keepdims=True)
    es = jnp.exp(s)
    sd = es / jnp.sum(es, axis=1, keepdims=True) + 1e-10
    t = (jnp.log(sd) + g_ref[...]) / t_ref[0, 0]
    t = t - jnp.max(t, axis=1, keepdims=True)
    et = jnp.exp(t)
    w = et / jnp.sum(et, axis=1, keepdims=True)
    eo = 1.0 / (1.0 + jnp.exp(-(logit * a_ref[...] + b_ref[...])))
    r = jnp.sum(eo * w, axis=1)
    out_ref[...] = jnp.clip(r, 0.0, 1.0)


def _run_head(u_emb, v_emb, g, lin_w, lin_b, sel_w, sel_b, a_prop, b_prop, t):
    n_blk = 8
    rows = B // n_blk
    full = lambda s: pl.BlockSpec(s, lambda i: (0,) * len(s))
    out = pl.pallas_call(
        _head_kernel,
        grid=(n_blk,),
        in_specs=[
            pl.BlockSpec((rows, EMB), lambda i: (i, 0)),
            pl.BlockSpec((rows, EMB), lambda i: (i, 0)),
            pl.BlockSpec((rows, E), lambda i: (i, 0)),
            full((EMB, 1)),
            full((EMB, 1)),
            full((1, 1)),
            full((EMB, E)),
            full((1, E)),
            full((1, E)),
            full((1, E)),
            full((1, 1)),
        ],
        out_specs=pl.BlockSpec((rows,), lambda i: (i,)),
        out_shape=jax.ShapeDtypeStruct((B,), jnp.float32),
    )(u_emb, v_emb, g, lin_w[:EMB], lin_w[EMB:], lin_b.reshape(1, 1),
      sel_w, sel_b.reshape(1, E), a_prop.reshape(1, E), b_prop.reshape(1, E),
      t)
    return out


def kernel(x, T, W_user, H_item, lin_w, lin_b, sel_w, sel_b, a_prop, b_prop, g):
    user_idx = x[:, 0].reshape(B // _CHUNK, _CHUNK)
    item_idx = x[:, 1].reshape(B // _CHUNK, _CHUNK)
    gather = _make_sc_gather(B, EMB)
    u_emb, v_emb = gather(user_idx, item_idx, W_user, H_item)
    t = jnp.asarray(T, jnp.float32).reshape(1, 1)
    return _run_head(u_emb, v_emb, g, lin_w, lin_b, sel_w, sel_b,
                     a_prop, b_prop, t)
